# baseline (speedup 1.0000x reference)
import functools

import jax
import jax.numpy as jnp
from jax import lax
from jax.experimental import pallas as pl
from jax.experimental.pallas import tpu as pltpu
from jax.experimental.pallas import tpu_sc as plsc

B = 16384
D = 64
H1 = 128
H2 = 64

NC = 2
NS = 16
NW = NC * NS
B_PER_W = B // NW


def _sc_gather(uid, iid, user_table, item_table):
    mesh = plsc.VectorSubcoreMesh(core_axis_name="c", subcore_axis_name="s")

    @functools.partial(
        pl.kernel,
        mesh=mesh,
        compiler_params=pltpu.CompilerParams(use_tc_tiling_on_sc=False),
        out_type=[
            jax.ShapeDtypeStruct((B, D), jnp.bfloat16),
            jax.ShapeDtypeStruct((B, D), jnp.bfloat16),
        ],
        scratch_types=[
            pltpu.VMEM((B_PER_W,), jnp.int32),
            pltpu.VMEM((B_PER_W,), jnp.int32),
            pltpu.VMEM((B_PER_W, D), jnp.bfloat16),
            pltpu.VMEM((B_PER_W, D), jnp.bfloat16),
            pltpu.SemaphoreType.DMA,
            pltpu.SemaphoreType.DMA,
        ],
    )
    def k(uid_hbm, iid_hbm, ut_hbm, it_hbm, uout_hbm, iout_hbm,
          uidx_v, iidx_v, urows_v, irows_v, sem_u, sem_i):
        wid = lax.axis_index("s") * NC + lax.axis_index("c")
        base = wid * B_PER_W
        pltpu.sync_copy(uid_hbm.at[pl.ds(base, B_PER_W)], uidx_v)
        pltpu.sync_copy(iid_hbm.at[pl.ds(base, B_PER_W)], iidx_v)
        cu = pltpu.async_copy(ut_hbm.at[uidx_v], urows_v, sem_u)
        ci = pltpu.async_copy(it_hbm.at[iidx_v], irows_v, sem_i)
        cu.wait()
        pltpu.sync_copy(urows_v, uout_hbm.at[pl.ds(base, B_PER_W)])
        ci.wait()
        pltpu.sync_copy(irows_v, iout_hbm.at[pl.ds(base, B_PER_W)])

    return k(uid, iid, user_table, item_table)


def _mlp_body(urows, irows, uw1, ub1, uw2, ub2, iw1, ib1, iw2, ib2, out):
    ur = urows[:].astype(jnp.float32)
    ir = irows[:].astype(jnp.float32)
    u = jnp.dot(ur, uw1[:], preferred_element_type=jnp.float32) + ub1[:]
    u = jnp.maximum(u, 0.0)
    u = jnp.dot(u, uw2[:], preferred_element_type=jnp.float32) + ub2[:]
    u = jnp.maximum(u, 0.0)
    v = jnp.dot(ir, iw1[:], preferred_element_type=jnp.float32) + ib1[:]
    v = jnp.maximum(v, 0.0)
    v = jnp.dot(v, iw2[:], preferred_element_type=jnp.float32) + ib2[:]
    v = jnp.maximum(v, 0.0)
    out[:] = jnp.maximum(jnp.sum(u * v, axis=1), 0.0).reshape(out.shape)


BLK = 1024


def _tc_mlp(urows, irows, uW1, ub1, uW2, ub2, iW1, ib1, iW2, ib2):
    nblk = B // BLK
    row_spec = pl.BlockSpec((BLK, D), lambda i: (i, 0))
    w1_spec = pl.BlockSpec((D, H1), lambda i: (0, 0))
    b1_spec = pl.BlockSpec((1, H1), lambda i: (0, 0))
    w2_spec = pl.BlockSpec((H1, H2), lambda i: (0, 0))
    b2_spec = pl.BlockSpec((1, H2), lambda i: (0, 0))
    out = pl.pallas_call(
        _mlp_body,
        grid=(nblk,),
        in_specs=[row_spec, row_spec,
                  w1_spec, b1_spec, w2_spec, b2_spec,
                  w1_spec, b1_spec, w2_spec, b2_spec],
        out_specs=pl.BlockSpec((BLK // 128, 128), lambda i: (i, 0)),
        out_shape=jax.ShapeDtypeStruct((B // 128, 128), jnp.float32),
    )(urows, irows,
      uW1, ub1.reshape(1, H1), uW2, ub2.reshape(1, H2),
      iW1, ib1.reshape(1, H1), iW2, ib2.reshape(1, H2))
    return out.reshape(-1)


def kernel(uid, iid, user_table, uW1, ub1, uW2, ub2, item_table, iW1, ib1, iW2, ib2):
    uid = uid.astype(jnp.int32)
    iid = iid.astype(jnp.int32)
    ut16 = user_table.astype(jnp.bfloat16)
    it16 = item_table.astype(jnp.bfloat16)
    urows, irows = _sc_gather(uid, iid, ut16, it16)
    return _tc_mlp(urows, irows, uW1, ub1, uW2, ub2, iW1, ib1, iW2, ib2)


# speedup vs baseline: 1.4433x; 1.4433x over previous
import functools

import jax
import jax.numpy as jnp
from jax import lax
from jax.experimental import pallas as pl
from jax.experimental.pallas import tpu as pltpu
from jax.experimental.pallas import tpu_sc as plsc

B = 16384
D = 64
H1 = 128
H2 = 64
HALF = 50176
PCOL = 256

NC = 2
NS = 16
NW = NC * NS
B_PER_W = B // NW


def _pack_body(ulo, uhi, ilo, ihi, uout, iout):
    uout[:] = jnp.concatenate(
        [jnp.transpose(ulo[:]), jnp.transpose(uhi[:])], axis=1)
    iout[:] = jnp.concatenate(
        [jnp.transpose(ilo[:]), jnp.transpose(ihi[:])], axis=1)


def _tc_pack(utT, itT):
    nblk = HALF // PCOL
    lo_spec = pl.BlockSpec((D, PCOL), lambda i: (0, i))
    hi_spec = pl.BlockSpec((D, PCOL), lambda i: (0, i + nblk))
    out_spec = pl.BlockSpec((PCOL, 2 * D), lambda i: (i, 0))
    out_shape = jax.ShapeDtypeStruct((HALF, 2 * D), jnp.float32)
    return pl.pallas_call(
        _pack_body,
        grid=(nblk,),
        in_specs=[lo_spec, hi_spec, lo_spec, hi_spec],
        out_specs=[out_spec, out_spec],
        out_shape=[out_shape, out_shape],
    )(utT, utT, itT, itT)


def _sc_gather(uidx2, iidx2, ut2, it2):
    mesh = plsc.VectorSubcoreMesh(core_axis_name="c", subcore_axis_name="s")

    @functools.partial(
        pl.kernel,
        mesh=mesh,
        compiler_params=pltpu.CompilerParams(use_tc_tiling_on_sc=True),
        out_type=[
            jax.ShapeDtypeStruct((B, 2 * D), jnp.float32),
            jax.ShapeDtypeStruct((B, 2 * D), jnp.float32),
        ],
        scratch_types=[
            pltpu.VMEM((B_PER_W // 2,), jnp.int32),
            pltpu.VMEM((B_PER_W // 2,), jnp.int32),
            pltpu.VMEM((B_PER_W // 2, 2 * D), jnp.float32),
            pltpu.VMEM((B_PER_W // 2, 2 * D), jnp.float32),
            pltpu.SemaphoreType.DMA,
            pltpu.SemaphoreType.DMA,
        ],
    )
    def k(uidx_hbm, iidx_hbm, ut_hbm, it_hbm, uout_hbm, iout_hbm,
          uidx_v, iidx_v, urows_v, irows_v, sem_u, sem_i):
        wid = lax.axis_index("s") * NC + lax.axis_index("c")
        cb = B_PER_W // 2
        for c in range(2):
            base = wid * B_PER_W + c * cb
            pltpu.sync_copy(uidx_hbm.at[pl.ds(base, cb)], uidx_v)
            pltpu.sync_copy(iidx_hbm.at[pl.ds(base, cb)], iidx_v)
            cu = pltpu.async_copy(ut_hbm.at[uidx_v], urows_v, sem_u)
            ci = pltpu.async_copy(it_hbm.at[iidx_v], irows_v, sem_i)
            cu.wait()
            pltpu.sync_copy(urows_v, uout_hbm.at[pl.ds(base, cb)])
            ci.wait()
            pltpu.sync_copy(irows_v, iout_hbm.at[pl.ds(base, cb)])

    return k(uidx2, iidx2, ut2, it2)


def _mlp_body(urows, irows, upar, ipar, uw1, ub1, uw2, ub2,
              iw1, ib1, iw2, ib2, out):
    ur = jnp.where(upar[:] == 0, urows[:, :D], urows[:, D:])
    ir = jnp.where(ipar[:] == 0, irows[:, :D], irows[:, D:])
    u = jnp.dot(ur, uw1[:], preferred_element_type=jnp.float32) + ub1[:]
    u = jnp.maximum(u, 0.0)
    u = jnp.dot(u, uw2[:], preferred_element_type=jnp.float32) + ub2[:]
    u = jnp.maximum(u, 0.0)
    v = jnp.dot(ir, iw1[:], preferred_element_type=jnp.float32) + ib1[:]
    v = jnp.maximum(v, 0.0)
    v = jnp.dot(v, iw2[:], preferred_element_type=jnp.float32) + ib2[:]
    v = jnp.maximum(v, 0.0)
    out[:] = jnp.maximum(jnp.sum(u * v, axis=1), 0.0).reshape(out.shape)


BLK = 1024


def _tc_mlp(urows, irows, upar, ipar, uW1, ub1, uW2, ub2, iW1, ib1, iW2, ib2):
    nblk = B // BLK
    row_spec = pl.BlockSpec((BLK, 2 * D), lambda i: (i, 0))
    par_spec = pl.BlockSpec((BLK, 1), lambda i: (i, 0))
    w1_spec = pl.BlockSpec((D, H1), lambda i: (0, 0))
    b1_spec = pl.BlockSpec((1, H1), lambda i: (0, 0))
    w2_spec = pl.BlockSpec((H1, H2), lambda i: (0, 0))
    b2_spec = pl.BlockSpec((1, H2), lambda i: (0, 0))
    out = pl.pallas_call(
        _mlp_body,
        grid=(nblk,),
        in_specs=[row_spec, row_spec, par_spec, par_spec,
                  w1_spec, b1_spec, w2_spec, b2_spec,
                  w1_spec, b1_spec, w2_spec, b2_spec],
        out_specs=pl.BlockSpec((BLK // 128, 128), lambda i: (i, 0)),
        out_shape=jax.ShapeDtypeStruct((B // 128, 128), jnp.float32),
    )(urows, irows, upar.reshape(B, 1), ipar.reshape(B, 1),
      uW1, ub1.reshape(1, H1), uW2, ub2.reshape(1, H2),
      iW1, ib1.reshape(1, H1), iW2, ib2.reshape(1, H2))
    return out.reshape(-1)


def kernel(uid, iid, user_table, uW1, ub1, uW2, ub2, item_table, iW1, ib1, iW2, ib2):
    uid = uid.astype(jnp.int32)
    iid = iid.astype(jnp.int32)
    ut2, it2 = _tc_pack(user_table.T, item_table.T)
    uidx2 = jnp.where(uid < HALF, uid, uid - HALF)
    iidx2 = jnp.where(iid < HALF, iid, iid - HALF)
    urows, irows = _sc_gather(uidx2, iidx2, ut2, it2)
    return _tc_mlp(urows, irows,
                   (uid >= HALF).astype(jnp.int32),
                   (iid >= HALF).astype(jnp.int32),
                   uW1, ub1, uW2, ub2, iW1, ib1, iW2, ib2)


# speedup vs baseline: 1.8440x; 1.2776x over previous
import functools

import jax
import jax.numpy as jnp
from jax import lax
from jax.experimental import pallas as pl
from jax.experimental.pallas import tpu as pltpu
from jax.experimental.pallas import tpu_sc as plsc

B = 16384
D = 64
H1 = 128
H2 = 64
HALF = 50176
PCOL = 512

NC = 2
NS = 16
NW = NC * NS
B_PER_W = B // NW


def _pack_body(ulo, uhi, ilo, ihi, eye, uout, iout):
    dn = (((0,), (0,)), ((), ()))
    f32 = jnp.float32
    tul = lax.dot_general(ulo[:], eye[:], dn, preferred_element_type=f32)
    tuh = lax.dot_general(uhi[:], eye[:], dn, preferred_element_type=f32)
    til = lax.dot_general(ilo[:], eye[:], dn, preferred_element_type=f32)
    tih = lax.dot_general(ihi[:], eye[:], dn, preferred_element_type=f32)
    uout[:] = jnp.concatenate([tul, tuh], axis=1)
    iout[:] = jnp.concatenate([til, tih], axis=1)


def _tc_pack(utT, itT):
    nblk = HALF // PCOL
    lo_spec = pl.BlockSpec((D, PCOL), lambda i: (0, i))
    hi_spec = pl.BlockSpec((D, PCOL), lambda i: (0, i + nblk))
    eye_spec = pl.BlockSpec((D, D), lambda i: (0, 0))
    out_spec = pl.BlockSpec((PCOL, 2 * D), lambda i: (i, 0))
    out_shape = jax.ShapeDtypeStruct((HALF, 2 * D), jnp.float32)
    return pl.pallas_call(
        _pack_body,
        grid=(nblk,),
        in_specs=[lo_spec, hi_spec, lo_spec, hi_spec, eye_spec],
        out_specs=[out_spec, out_spec],
        out_shape=[out_shape, out_shape],
    )(utT, utT, itT, itT, jnp.eye(D, dtype=jnp.float32))


def _sc_gather(uidx2, iidx2, ut2, it2):
    mesh = plsc.VectorSubcoreMesh(core_axis_name="c", subcore_axis_name="s")

    @functools.partial(
        pl.kernel,
        mesh=mesh,
        compiler_params=pltpu.CompilerParams(use_tc_tiling_on_sc=True),
        out_type=[
            jax.ShapeDtypeStruct((B, 2 * D), jnp.float32),
            jax.ShapeDtypeStruct((B, 2 * D), jnp.float32),
        ],
        scratch_types=[
            pltpu.VMEM((B_PER_W // 2,), jnp.int32),
            pltpu.VMEM((B_PER_W // 2,), jnp.int32),
            pltpu.VMEM((B_PER_W // 2, 2 * D), jnp.float32),
            pltpu.VMEM((B_PER_W // 2, 2 * D), jnp.float32),
            pltpu.SemaphoreType.DMA,
            pltpu.SemaphoreType.DMA,
        ],
    )
    def k(uidx_hbm, iidx_hbm, ut_hbm, it_hbm, uout_hbm, iout_hbm,
          uidx_v, iidx_v, urows_v, irows_v, sem_u, sem_i):
        wid = lax.axis_index("s") * NC + lax.axis_index("c")
        cb = B_PER_W // 2
        for c in range(2):
            base = wid * B_PER_W + c * cb
            pltpu.sync_copy(uidx_hbm.at[pl.ds(base, cb)], uidx_v)
            pltpu.sync_copy(iidx_hbm.at[pl.ds(base, cb)], iidx_v)
            cu = pltpu.async_copy(ut_hbm.at[uidx_v], urows_v, sem_u)
            ci = pltpu.async_copy(it_hbm.at[iidx_v], irows_v, sem_i)
            cu.wait()
            pltpu.sync_copy(urows_v, uout_hbm.at[pl.ds(base, cb)])
            ci.wait()
            pltpu.sync_copy(irows_v, iout_hbm.at[pl.ds(base, cb)])

    return k(uidx2, iidx2, ut2, it2)


def _mlp_body(urows, irows, upar, ipar, uw1, ub1, uw2, ub2,
              iw1, ib1, iw2, ib2, out):
    ur = jnp.where(upar[:] == 0, urows[:, :D], urows[:, D:])
    ir = jnp.where(ipar[:] == 0, irows[:, :D], irows[:, D:])
    u = jnp.dot(ur, uw1[:], preferred_element_type=jnp.float32) + ub1[:]
    u = jnp.maximum(u, 0.0)
    u = jnp.dot(u, uw2[:], preferred_element_type=jnp.float32) + ub2[:]
    u = jnp.maximum(u, 0.0)
    v = jnp.dot(ir, iw1[:], preferred_element_type=jnp.float32) + ib1[:]
    v = jnp.maximum(v, 0.0)
    v = jnp.dot(v, iw2[:], preferred_element_type=jnp.float32) + ib2[:]
    v = jnp.maximum(v, 0.0)
    out[:] = jnp.maximum(jnp.sum(u * v, axis=1), 0.0).reshape(out.shape)


BLK = 1024


def _tc_mlp(urows, irows, upar, ipar, uW1, ub1, uW2, ub2, iW1, ib1, iW2, ib2):
    nblk = B // BLK
    row_spec = pl.BlockSpec((BLK, 2 * D), lambda i: (i, 0))
    par_spec = pl.BlockSpec((BLK, 1), lambda i: (i, 0))
    w1_spec = pl.BlockSpec((D, H1), lambda i: (0, 0))
    b1_spec = pl.BlockSpec((1, H1), lambda i: (0, 0))
    w2_spec = pl.BlockSpec((H1, H2), lambda i: (0, 0))
    b2_spec = pl.BlockSpec((1, H2), lambda i: (0, 0))
    out = pl.pallas_call(
        _mlp_body,
        grid=(nblk,),
        in_specs=[row_spec, row_spec, par_spec, par_spec,
                  w1_spec, b1_spec, w2_spec, b2_spec,
                  w1_spec, b1_spec, w2_spec, b2_spec],
        out_specs=pl.BlockSpec((BLK // 128, 128), lambda i: (i, 0)),
        out_shape=jax.ShapeDtypeStruct((B // 128, 128), jnp.float32),
    )(urows, irows, upar.reshape(B, 1), ipar.reshape(B, 1),
      uW1, ub1.reshape(1, H1), uW2, ub2.reshape(1, H2),
      iW1, ib1.reshape(1, H1), iW2, ib2.reshape(1, H2))
    return out.reshape(-1)


def kernel(uid, iid, user_table, uW1, ub1, uW2, ub2, item_table, iW1, ib1, iW2, ib2):
    uid = uid.astype(jnp.int32)
    iid = iid.astype(jnp.int32)
    ut2, it2 = _tc_pack(user_table.T, item_table.T)
    uidx2 = jnp.where(uid < HALF, uid, uid - HALF)
    iidx2 = jnp.where(iid < HALF, iid, iid - HALF)
    urows, irows = _sc_gather(uidx2, iidx2, ut2, it2)
    return _tc_mlp(urows, irows,
                   (uid >= HALF).astype(jnp.int32),
                   (iid >= HALF).astype(jnp.int32),
                   uW1, ub1, uW2, ub2, iW1, ib1, iW2, ib2)
